# Initial kernel scaffold; baseline (speedup 1.0000x reference)
import functools

import jax
import jax.numpy as jnp
from jax import lax
from jax.experimental import pallas as pl
from jax.experimental.pallas import tpu as pltpu
from jax.experimental.pallas import tpu_sc as plsc

N = 10000
E = 320000
D = 128
OUT = 128

NC = 2
NS = 16
NW = NC * NS

CHUNK = 80
EPW = E // NW
NCHUNK = EPW // CHUNK
ROWS_PT = N // NS
ZROWS = 125


def _sc_agg_body(h_hbm, src_hbm, dst_hbm, w_hbm, out_hbm,
                 src_v, dst_v, w_v, rows_v, zero_v, agg_sh):
    cid = lax.axis_index("c")
    sid = lax.axis_index("s")
    wid = cid * NS + sid

    pltpu.sync_copy(src_hbm.at[wid], src_v)
    pltpu.sync_copy(dst_hbm.at[wid], dst_v)
    pltpu.sync_copy(w_hbm.at[wid], w_v)

    def zero_row(r, _):
        z = jnp.zeros((16,), jnp.float32)
        for u in range(D // 16):
            zero_v[r, pl.ds(u * 16, 16)] = z
        return 0
    lax.fori_loop(0, ZROWS, zero_row, 0)
    for t in range(ROWS_PT // ZROWS):
        pltpu.sync_copy(zero_v, agg_sh.at[pl.ds(sid * ROWS_PT + t * ZROWS, ZROWS)])
    plsc.subcore_barrier()

    def do_chunk(j, _):
        pltpu.sync_copy(h_hbm.at[src_v.at[j]], rows_v)
        jf = jnp.full((16,), j, jnp.int32)
        def scale_row(r, _):
            wv = plsc.load_gather(w_v, [jf, jnp.full((16,), r, jnp.int32)])
            for u in range(D // 16):
                rows_v[r, pl.ds(u * 16, 16)] = rows_v[r, pl.ds(u * 16, 16)] * wv
            return 0
        lax.fori_loop(0, CHUNK, scale_row, 0)
        pltpu.sync_copy(rows_v, agg_sh.at[dst_v.at[j]], add=True)
        return 0
    lax.fori_loop(0, NCHUNK, do_chunk, 0)

    plsc.subcore_barrier()
    pltpu.sync_copy(agg_sh.at[pl.ds(sid * ROWS_PT, ROWS_PT)],
                    out_hbm.at[cid, pl.ds(sid * ROWS_PT, ROWS_PT)])


@jax.jit
def _sc_aggregate(h, src, dst, w):
    mesh = plsc.VectorSubcoreMesh(core_axis_name="c", subcore_axis_name="s")
    return pl.kernel(
        _sc_agg_body,
        out_type=jax.ShapeDtypeStruct((NC, N, D), jnp.float32),
        mesh=mesh,
        scratch_types=[
            pltpu.VMEM((NCHUNK, CHUNK), jnp.int32),
            pltpu.VMEM((NCHUNK, CHUNK), jnp.int32),
            pltpu.VMEM((NCHUNK, CHUNK), jnp.float32),
            pltpu.VMEM((CHUNK, D), jnp.float32),
            pltpu.VMEM((ZROWS, D), jnp.float32),
            pltpu.VMEM_SHARED((N, D), jnp.float32),
        ],
    )(h, src, dst, w)


ROWB = 400


def _linear_body(h_b, p0_b, p1_b, w1t_b, w2t_b, b_b, out_b):
    agg = p0_b[...] + p1_b[...]
    acc = jnp.dot(h_b[...], w1t_b[...], preferred_element_type=jnp.float32)
    acc += jnp.dot(agg, w2t_b[...], preferred_element_type=jnp.float32)
    out_b[...] = acc + b_b[...]


@jax.jit
def _linear(h, p0, p1, w1t, w2t, b2d):
    grid = (N // ROWB,)
    return pl.pallas_call(
        _linear_body,
        grid=grid,
        in_specs=[
            pl.BlockSpec((ROWB, D), lambda i: (i, 0)),
            pl.BlockSpec((ROWB, D), lambda i: (i, 0)),
            pl.BlockSpec((ROWB, D), lambda i: (i, 0)),
            pl.BlockSpec((D, OUT), lambda i: (0, 0)),
            pl.BlockSpec((D, OUT), lambda i: (0, 0)),
            pl.BlockSpec((1, OUT), lambda i: (0, 0)),
        ],
        out_specs=pl.BlockSpec((ROWB, OUT), lambda i: (i, 0)),
        out_shape=jax.ShapeDtypeStruct((N, OUT), jnp.float32),
    )(h, p0, p1, w1t, w2t, b2d)


def kernel(h, edge_index, edge_w, W, b):
    src = edge_index[0].astype(jnp.int32).reshape(NW, NCHUNK, CHUNK)
    dst = edge_index[1].astype(jnp.int32).reshape(NW, NCHUNK, CHUNK)
    w = edge_w.reshape(NW, NCHUNK, CHUNK).astype(jnp.float32)
    partials = _sc_aggregate(h, src, dst, w)
    w1t = W[:, :D].T
    w2t = W[:, D:].T
    return _linear(h, partials[0], partials[1], w1t, w2t, b.reshape(1, OUT))


# speedup vs baseline: 6.2760x; 6.2760x over previous
import functools

import jax
import jax.numpy as jnp
from jax import lax
from jax.experimental import pallas as pl
from jax.experimental.pallas import tpu as pltpu
from jax.experimental.pallas import tpu_sc as plsc

N = 10000
E = 320000
D = 128
OUT = 128

NC = 2
NS = 16
NW = NC * NS

CHUNK = 80
EPW = E // NW
NCHUNK = EPW // CHUNK
NPAD = 10240
ROWS_PT = NPAD // NS


def _sc_agg_body(h_hbm, src_hbm, dst_hbm, w_hbm, out_hbm,
                 src_v, dst_v, w_v, rows_v, agg_sh):
    cid = lax.axis_index("c")
    sid = lax.axis_index("s")
    wid = cid * NS + sid

    pltpu.sync_copy(src_hbm.at[wid], src_v)
    pltpu.sync_copy(dst_hbm.at[wid], dst_v)
    pltpu.sync_copy(w_hbm.at[wid], w_v)

    def zero_row(r, _):
        z = jnp.zeros((16,), jnp.float32)
        for u in range(D // 16):
            rows_v[r, pl.ds(u * 16, 16)] = z
        return 0
    lax.fori_loop(0, CHUNK, zero_row, 0)
    for t in range(ROWS_PT // CHUNK):
        pltpu.sync_copy(rows_v, agg_sh.at[pl.ds(sid * ROWS_PT + t * CHUNK, CHUNK)])
    plsc.subcore_barrier()

    def do_chunk(j, _):
        pltpu.sync_copy(h_hbm.at[src_v.at[pl.ds(j * CHUNK, CHUNK)]], rows_v)
        def scale_group(g, _):
            w16 = w_v[pl.ds(j * CHUNK + g * 16, 16)]
            for l in range(16):
                wv = lax.gather(
                    w16, jnp.full((16, 1), l, jnp.int32),
                    lax.GatherDimensionNumbers(offset_dims=(),
                                               collapsed_slice_dims=(0,),
                                               start_index_map=(0,)),
                    (1,), mode=lax.GatherScatterMode.PROMISE_IN_BOUNDS)
                r = g * 16 + l
                for u in range(D // 16):
                    rows_v[r, pl.ds(u * 16, 16)] = rows_v[r, pl.ds(u * 16, 16)] * wv
            return 0
        lax.fori_loop(0, CHUNK // 16, scale_group, 0)
        pltpu.sync_copy(rows_v, agg_sh.at[dst_v.at[j]], add=True)
        return 0
    lax.fori_loop(0, NCHUNK, do_chunk, 0)

    plsc.subcore_barrier()
    pltpu.sync_copy(agg_sh.at[pl.ds(sid * ROWS_PT, ROWS_PT)],
                    out_hbm.at[cid, pl.ds(sid * ROWS_PT, ROWS_PT)])


@jax.jit
def _sc_aggregate(h, src, dst, w):
    mesh = plsc.VectorSubcoreMesh(core_axis_name="c", subcore_axis_name="s")
    return pl.kernel(
        _sc_agg_body,
        out_type=jax.ShapeDtypeStruct((NC, NPAD, D), jnp.float32),
        mesh=mesh,
        scratch_types=[
            pltpu.VMEM((EPW,), jnp.int32),
            pltpu.VMEM((NCHUNK, CHUNK), jnp.int32),
            pltpu.VMEM((EPW,), jnp.float32),
            pltpu.VMEM((CHUNK, D), jnp.float32),
            pltpu.VMEM_SHARED((NPAD, D), jnp.float32),
        ],
    )(h, src, dst, w)


ROWB = 400


def _linear_body(h_b, p0_b, p1_b, w1t_b, w2t_b, b_b, out_b):
    agg = p0_b[...] + p1_b[...]
    acc = jnp.dot(h_b[...], w1t_b[...], preferred_element_type=jnp.float32)
    acc += jnp.dot(agg, w2t_b[...], preferred_element_type=jnp.float32)
    out_b[...] = acc + b_b[...]


@jax.jit
def _linear(h, p0, p1, w1t, w2t, b2d):
    grid = (N // ROWB,)
    return pl.pallas_call(
        _linear_body,
        grid=grid,
        in_specs=[
            pl.BlockSpec((ROWB, D), lambda i: (i, 0)),
            pl.BlockSpec((ROWB, D), lambda i: (i, 0)),
            pl.BlockSpec((ROWB, D), lambda i: (i, 0)),
            pl.BlockSpec((D, OUT), lambda i: (0, 0)),
            pl.BlockSpec((D, OUT), lambda i: (0, 0)),
            pl.BlockSpec((1, OUT), lambda i: (0, 0)),
        ],
        out_specs=pl.BlockSpec((ROWB, OUT), lambda i: (i, 0)),
        out_shape=jax.ShapeDtypeStruct((N, OUT), jnp.float32),
    )(h, p0, p1, w1t, w2t, b2d)


def kernel(h, edge_index, edge_w, W, b):
    src = edge_index[0].astype(jnp.int32).reshape(NW, EPW)
    dst = edge_index[1].astype(jnp.int32).reshape(NW, NCHUNK, CHUNK)
    w = edge_w.reshape(NW, EPW).astype(jnp.float32)
    partials = _sc_aggregate(h, src, dst, w)[:, :N, :]
    w1t = W[:, :D].T
    w2t = W[:, D:].T
    return _linear(h, partials[0], partials[1], w1t, w2t, b.reshape(1, OUT))


# speedup vs baseline: 9.5122x; 1.5156x over previous
import jax
import jax.numpy as jnp
from jax import lax
from jax.experimental import pallas as pl
from jax.experimental.pallas import tpu as pltpu
from jax.experimental.pallas import tpu_sc as plsc

N = 10000
E = 320000
D = 128
OUT = 128

NC = 2
NS = 16
NW = NC * NS

CHUNK = 80
EPW = E // NW
NCHUNK = EPW // CHUNK
NPAD = 10240
ROWS_PT = NPAD // NS


def _scale_rows(buf, wc):
    def scale_group(g, _):
        w16 = wc[pl.ds(g * 16, 16)]
        for l in range(16):
            wv = lax.gather(
                w16, jnp.full((16, 1), l, jnp.int32),
                lax.GatherDimensionNumbers(offset_dims=(),
                                           collapsed_slice_dims=(0,),
                                           start_index_map=(0,)),
                (1,), mode=lax.GatherScatterMode.PROMISE_IN_BOUNDS)
            for u in range(D // 16):
                buf[g * 16 + l, pl.ds(u * 16, 16)] = (
                    buf[g * 16 + l, pl.ds(u * 16, 16)] * wv)
        return 0
    lax.fori_loop(0, CHUNK // 16, scale_group, 0)


def _sc_agg_body(h_hbm, src_hbm, dst_hbm, w_hbm, out_hbm,
                 src_v, dst_v, wc_v, buf_v, agg_sh, sems, wsems):
    cid = lax.axis_index("c")
    sid = lax.axis_index("s")
    wid = cid * NS + sid

    def start_fetch(j, b):
        pltpu.async_copy(w_hbm.at[wid, j], wc_v.at[b], wsems.at[b])
        pltpu.async_copy(h_hbm.at[src_v.at[pl.ds(j * CHUNK, CHUNK)]],
                         buf_v.at[b], sems.at[b])

    def wait_fetch(j, b):
        pltpu.make_async_copy(w_hbm.at[wid, j], wc_v.at[b], wsems.at[b]).wait()
        pltpu.make_async_copy(h_hbm.at[src_v.at[pl.ds(j * CHUNK, CHUNK)]],
                              buf_v.at[b], sems.at[b]).wait()

    pltpu.sync_copy(src_hbm.at[wid], src_v)
    start_fetch(0, 0)
    pltpu.sync_copy(dst_hbm.at[wid], dst_v)

    def zero_row(r, _):
        z = jnp.zeros((16,), jnp.float32)
        for u in range(D // 16):
            buf_v[1, r, pl.ds(u * 16, 16)] = z
        return 0
    lax.fori_loop(0, CHUNK, zero_row, 0)
    for t in range(ROWS_PT // CHUNK):
        pltpu.sync_copy(buf_v.at[1],
                        agg_sh.at[pl.ds(sid * ROWS_PT + t * CHUNK, CHUNK)])
    plsc.subcore_barrier()

    def process(j, b):
        wait_fetch(j, b)
        _scale_rows(buf_v.at[b], wc_v.at[b, 0])
        pltpu.sync_copy(buf_v.at[b], agg_sh.at[dst_v.at[j]], add=True)

    def do_pair(jj, _):
        for b in range(2):
            j = jj * 2 + b
            start_fetch(j + 1, 1 - b)
            process(j, b)
        return 0
    lax.fori_loop(0, (NCHUNK - 1) // 2, do_pair, 0)
    process(NCHUNK - 1, 0)

    plsc.subcore_barrier()
    pltpu.sync_copy(agg_sh.at[pl.ds(sid * ROWS_PT, ROWS_PT)],
                    out_hbm.at[cid, pl.ds(sid * ROWS_PT, ROWS_PT)])


@jax.jit
def _sc_aggregate(h, src, dst, w):
    mesh = plsc.VectorSubcoreMesh(core_axis_name="c", subcore_axis_name="s")
    return pl.kernel(
        _sc_agg_body,
        out_type=jax.ShapeDtypeStruct((NC, NPAD, D), jnp.float32),
        mesh=mesh,
        scratch_types=[
            pltpu.VMEM((EPW,), jnp.int32),
            pltpu.VMEM((NCHUNK, CHUNK), jnp.int32),
            pltpu.VMEM((2, 1, CHUNK), jnp.float32),
            pltpu.VMEM((2, CHUNK, D), jnp.float32),
            pltpu.VMEM_SHARED((NPAD, D), jnp.float32),
            pltpu.SemaphoreType.DMA((2,)),
            pltpu.SemaphoreType.DMA((2,)),
        ],
    )(h, src, dst, w)


ROWB = 400


def _linear_body(h_b, p0_b, p1_b, w1t_b, w2t_b, b_b, out_b):
    agg = p0_b[...] + p1_b[...]
    acc = jnp.dot(h_b[...], w1t_b[...], preferred_element_type=jnp.float32)
    acc += jnp.dot(agg, w2t_b[...], preferred_element_type=jnp.float32)
    out_b[...] = acc + b_b[...]


@jax.jit
def _linear(h, p0, p1, w1t, w2t, b2d):
    grid = (N // ROWB,)
    return pl.pallas_call(
        _linear_body,
        grid=grid,
        in_specs=[
            pl.BlockSpec((ROWB, D), lambda i: (i, 0)),
            pl.BlockSpec((ROWB, D), lambda i: (i, 0)),
            pl.BlockSpec((ROWB, D), lambda i: (i, 0)),
            pl.BlockSpec((D, OUT), lambda i: (0, 0)),
            pl.BlockSpec((D, OUT), lambda i: (0, 0)),
            pl.BlockSpec((1, OUT), lambda i: (0, 0)),
        ],
        out_specs=pl.BlockSpec((ROWB, OUT), lambda i: (i, 0)),
        out_shape=jax.ShapeDtypeStruct((N, OUT), jnp.float32),
    )(h, p0, p1, w1t, w2t, b2d)


def kernel(h, edge_index, edge_w, W, b):
    src = edge_index[0].astype(jnp.int32).reshape(NW, EPW)
    dst = edge_index[1].astype(jnp.int32).reshape(NW, NCHUNK, CHUNK)
    w = edge_w.reshape(NW, NCHUNK, 1, CHUNK).astype(jnp.float32)
    partials = _sc_aggregate(h, src, dst, w)[:, :N, :]
    w1t = W[:, :D].T
    w2t = W[:, D:].T
    return _linear(h, partials[0], partials[1], w1t, w2t, b.reshape(1, OUT))


# speedup vs baseline: 9.5277x; 1.0016x over previous
import jax
import jax.numpy as jnp
from jax import lax
from jax.experimental import pallas as pl
from jax.experimental.pallas import tpu as pltpu
from jax.experimental.pallas import tpu_sc as plsc

N = 10000
E = 320000
D = 128
OUT = 128

NC = 2
NS = 16
NW = NC * NS

CHUNK = 80
EPW = E // NW
NCHUNK = EPW // CHUNK
NPAD = 10240
ROWS_PT = NPAD // NS


def _scale_rows(buf, wc):
    def scale_group(g, _):
        w16 = wc[pl.ds(g * 16, 16)]
        for l in range(16):
            wv = lax.gather(
                w16, jnp.full((16, 1), l, jnp.int32),
                lax.GatherDimensionNumbers(offset_dims=(),
                                           collapsed_slice_dims=(0,),
                                           start_index_map=(0,)),
                (1,), mode=lax.GatherScatterMode.PROMISE_IN_BOUNDS)
            for u in range(D // 16):
                buf[g * 16 + l, pl.ds(u * 16, 16)] = (
                    buf[g * 16 + l, pl.ds(u * 16, 16)] * wv)
        return 0
    lax.fori_loop(0, CHUNK // 16, scale_group, 0)


def _sc_agg_body(h_hbm, src_hbm, dst_hbm, w_hbm, out_hbm,
                 src_v, dst_v, wc_v, buf_v, agg_sh, sems, wsems, ssems):
    cid = lax.axis_index("c")
    sid = lax.axis_index("s")
    wid = cid * NS + sid

    def start_fetch(j, b):
        pltpu.async_copy(w_hbm.at[wid, j], wc_v.at[b], wsems.at[b])
        pltpu.async_copy(h_hbm.at[src_v.at[pl.ds(j * CHUNK, CHUNK)]],
                         buf_v.at[b], sems.at[b])

    def wait_fetch(j, b):
        pltpu.make_async_copy(w_hbm.at[wid, j], wc_v.at[b], wsems.at[b]).wait()
        pltpu.make_async_copy(h_hbm.at[src_v.at[pl.ds(j * CHUNK, CHUNK)]],
                              buf_v.at[b], sems.at[b]).wait()

    pltpu.sync_copy(src_hbm.at[wid], src_v)
    start_fetch(0, 0)
    pltpu.sync_copy(dst_hbm.at[wid], dst_v)

    def zero_row(r, _):
        z = jnp.zeros((16,), jnp.float32)
        for u in range(D // 16):
            buf_v[1, r, pl.ds(u * 16, 16)] = z
        return 0
    lax.fori_loop(0, CHUNK, zero_row, 0)
    for t in range(ROWS_PT // CHUNK):
        pltpu.sync_copy(buf_v.at[1],
                        agg_sh.at[pl.ds(sid * ROWS_PT + t * CHUNK, CHUNK)])
    plsc.subcore_barrier()

    def wait_scatter(j, b):
        pltpu.make_async_copy(buf_v.at[b], agg_sh.at[dst_v.at[j]],
                              ssems.at[b]).wait()

    def process(j, b):
        wait_fetch(j, b)
        _scale_rows(buf_v.at[b], wc_v.at[b, 0])
        pltpu.async_copy(buf_v.at[b], agg_sh.at[dst_v.at[j]], ssems.at[b],
                         add=True)

    def do_pair(jj, _):
        for b in range(2):
            j = jj * 2 + b

            @pl.when(j >= 1)
            def _():
                wait_scatter(j - 1, 1 - b)

            start_fetch(j + 1, 1 - b)
            process(j, b)
        return 0
    lax.fori_loop(0, (NCHUNK - 1) // 2, do_pair, 0)
    wait_scatter(NCHUNK - 2, 1)
    process(NCHUNK - 1, 0)
    wait_scatter(NCHUNK - 1, 0)

    plsc.subcore_barrier()
    pltpu.sync_copy(agg_sh.at[pl.ds(sid * ROWS_PT, ROWS_PT)],
                    out_hbm.at[cid, pl.ds(sid * ROWS_PT, ROWS_PT)])


@jax.jit
def _sc_aggregate(h, src, dst, w):
    mesh = plsc.VectorSubcoreMesh(core_axis_name="c", subcore_axis_name="s")
    return pl.kernel(
        _sc_agg_body,
        out_type=jax.ShapeDtypeStruct((NC, NPAD, D), jnp.float32),
        mesh=mesh,
        scratch_types=[
            pltpu.VMEM((EPW,), jnp.int32),
            pltpu.VMEM((NCHUNK, CHUNK), jnp.int32),
            pltpu.VMEM((2, 1, CHUNK), jnp.float32),
            pltpu.VMEM((2, CHUNK, D), jnp.float32),
            pltpu.VMEM_SHARED((NPAD, D), jnp.float32),
            pltpu.SemaphoreType.DMA((2,)),
            pltpu.SemaphoreType.DMA((2,)),
            pltpu.SemaphoreType.DMA((2,)),
        ],
    )(h, src, dst, w)


ROWB = 400


def _linear_body(h_b, p0_b, p1_b, w1t_b, w2t_b, b_b, out_b):
    agg = p0_b[...] + p1_b[...]
    acc = jnp.dot(h_b[...], w1t_b[...], preferred_element_type=jnp.float32)
    acc += jnp.dot(agg, w2t_b[...], preferred_element_type=jnp.float32)
    out_b[...] = acc + b_b[...]


@jax.jit
def _linear(h, p0, p1, w1t, w2t, b2d):
    grid = (N // ROWB,)
    return pl.pallas_call(
        _linear_body,
        grid=grid,
        in_specs=[
            pl.BlockSpec((ROWB, D), lambda i: (i, 0)),
            pl.BlockSpec((ROWB, D), lambda i: (i, 0)),
            pl.BlockSpec((ROWB, D), lambda i: (i, 0)),
            pl.BlockSpec((D, OUT), lambda i: (0, 0)),
            pl.BlockSpec((D, OUT), lambda i: (0, 0)),
            pl.BlockSpec((1, OUT), lambda i: (0, 0)),
        ],
        out_specs=pl.BlockSpec((ROWB, OUT), lambda i: (i, 0)),
        out_shape=jax.ShapeDtypeStruct((N, OUT), jnp.float32),
    )(h, p0, p1, w1t, w2t, b2d)


def kernel(h, edge_index, edge_w, W, b):
    src = edge_index[0].astype(jnp.int32).reshape(NW, EPW)
    dst = edge_index[1].astype(jnp.int32).reshape(NW, NCHUNK, CHUNK)
    w = edge_w.reshape(NW, NCHUNK, 1, CHUNK).astype(jnp.float32)
    partials = _sc_aggregate(h, src, dst, w)[:, :N, :]
    w1t = W[:, :D].T
    w2t = W[:, D:].T
    return _linear(h, partials[0], partials[1], w1t, w2t, b.reshape(1, OUT))


# speedup vs baseline: 10.4082x; 1.0924x over previous
import jax
import jax.numpy as jnp
from jax import lax
from jax.experimental import pallas as pl
from jax.experimental.pallas import tpu as pltpu
from jax.experimental.pallas import tpu_sc as plsc

N = 10000
E = 320000
D = 128
OUT = 128

NC = 2
NS = 16
NW = NC * NS

NBUF = 3
CHUNK = 80
EPW = E // NW
NCHUNK = EPW // CHUNK
NPAD = 10240
ROWS_PT = NPAD // NS


def _scale_rows(buf, wc):
    def scale_group(g, _):
        w16 = wc[pl.ds(g * 16, 16)]
        for l in range(16):
            wv = lax.gather(
                w16, jnp.full((16, 1), l, jnp.int32),
                lax.GatherDimensionNumbers(offset_dims=(),
                                           collapsed_slice_dims=(0,),
                                           start_index_map=(0,)),
                (1,), mode=lax.GatherScatterMode.PROMISE_IN_BOUNDS)
            r = g * 16 + l
            for u in range(D // 16):
                buf[r, pl.ds(u * 16, 16)] = buf[r, pl.ds(u * 16, 16)] * wv
        return 0
    lax.fori_loop(0, CHUNK // 16, scale_group, 0)


def _sc_agg_body(h_hbm, src_hbm, dst_hbm, w_hbm, out_hbm,
                 src_v, dst_c, wc_v, buf_v, agg_sh,
                 sems, wsems, dsems, ssems, zsem):
    cid = lax.axis_index("c")
    sid = lax.axis_index("s")
    wid = cid * NS + sid

    def start_fetch(j, b):
        pltpu.async_copy(w_hbm.at[wid, j], wc_v.at[b], wsems.at[b])
        pltpu.async_copy(dst_hbm.at[wid, j], dst_c.at[b], dsems.at[b])
        pltpu.async_copy(h_hbm.at[src_v.at[pl.ds(j * CHUNK, CHUNK)]],
                         buf_v.at[b], sems.at[b])

    def wait_fetch(j, b):
        pltpu.make_async_copy(w_hbm.at[wid, j], wc_v.at[b], wsems.at[b]).wait()
        pltpu.make_async_copy(dst_hbm.at[wid, j], dst_c.at[b],
                              dsems.at[b]).wait()
        pltpu.make_async_copy(h_hbm.at[src_v.at[pl.ds(j * CHUNK, CHUNK)]],
                              buf_v.at[b], sems.at[b]).wait()

    def start_scatter(b):
        pltpu.async_copy(buf_v.at[b], agg_sh.at[dst_c.at[b, 0]], ssems.at[b],
                         add=True)

    def wait_scatter(b):
        pltpu.make_async_copy(buf_v.at[b], agg_sh.at[dst_c.at[b, 0]],
                              ssems.at[b]).wait()

    pltpu.sync_copy(src_hbm.at[wid], src_v)
    start_fetch(0, 0)
    start_fetch(1, 1)

    def zero_row(r, _):
        z = jnp.zeros((16,), jnp.float32)
        for u in range(D // 16):
            buf_v[2, r, pl.ds(u * 16, 16)] = z
        return 0
    lax.fori_loop(0, CHUNK, zero_row, 0)
    for t in range(ROWS_PT // CHUNK):
        pltpu.async_copy(buf_v.at[2],
                         agg_sh.at[pl.ds(sid * ROWS_PT + t * CHUNK, CHUNK)],
                         zsem)
    for t in range(ROWS_PT // CHUNK):
        pltpu.make_async_copy(buf_v.at[2],
                              agg_sh.at[pl.ds(sid * ROWS_PT + t * CHUNK,
                                              CHUNK)],
                              zsem).wait()
    plsc.subcore_barrier()

    def step(j, b, fetch_ahead, drain_prev):
        wait_fetch(j, b)
        _scale_rows(buf_v.at[b], wc_v.at[b, 0])
        if drain_prev:
            wait_scatter((b + 2) % NBUF)
        if fetch_ahead:
            start_fetch(j + 2, (b + 2) % NBUF)
        start_scatter(b)

    def do_triple(jt, _):
        for b in range(NBUF):
            j = jt * NBUF + b
            wait_fetch(j, b)
            _scale_rows(buf_v.at[b], wc_v.at[b, 0])

            @pl.when(j >= 1)
            def _():
                wait_scatter((b + 2) % NBUF)
            start_fetch(j + 2, (b + 2) % NBUF)
            start_scatter(b)
        return 0
    lax.fori_loop(0, (NCHUNK - 2) // NBUF, do_triple, 0)
    step(NCHUNK - 2, 0, False, True)
    step(NCHUNK - 1, 1, False, True)
    wait_scatter(1)

    plsc.subcore_barrier()
    pltpu.sync_copy(agg_sh.at[pl.ds(sid * ROWS_PT, ROWS_PT)],
                    out_hbm.at[cid, pl.ds(sid * ROWS_PT, ROWS_PT)])


@jax.jit
def _sc_aggregate(h, src, dst, w):
    mesh = plsc.VectorSubcoreMesh(core_axis_name="c", subcore_axis_name="s")
    return pl.kernel(
        _sc_agg_body,
        out_type=jax.ShapeDtypeStruct((NC, NPAD, D), jnp.float32),
        mesh=mesh,
        scratch_types=[
            pltpu.VMEM((EPW,), jnp.int32),
            pltpu.VMEM((NBUF, 1, CHUNK), jnp.int32),
            pltpu.VMEM((NBUF, 1, CHUNK), jnp.float32),
            pltpu.VMEM((NBUF, CHUNK, D), jnp.float32),
            pltpu.VMEM_SHARED((NPAD, D), jnp.float32),
            pltpu.SemaphoreType.DMA((NBUF,)),
            pltpu.SemaphoreType.DMA((NBUF,)),
            pltpu.SemaphoreType.DMA((NBUF,)),
            pltpu.SemaphoreType.DMA((NBUF,)),
            pltpu.SemaphoreType.DMA,
        ],
    )(h, src, dst, w)


ROWB = 400


def _linear_body(h_b, p0_b, p1_b, w1t_b, w2t_b, b_b, out_b):
    agg = p0_b[...] + p1_b[...]
    acc = jnp.dot(h_b[...], w1t_b[...], preferred_element_type=jnp.float32)
    acc += jnp.dot(agg, w2t_b[...], preferred_element_type=jnp.float32)
    out_b[...] = acc + b_b[...]


@jax.jit
def _linear(h, p0, p1, w1t, w2t, b2d):
    grid = (N // ROWB,)
    return pl.pallas_call(
        _linear_body,
        grid=grid,
        in_specs=[
            pl.BlockSpec((ROWB, D), lambda i: (i, 0)),
            pl.BlockSpec((ROWB, D), lambda i: (i, 0)),
            pl.BlockSpec((ROWB, D), lambda i: (i, 0)),
            pl.BlockSpec((D, OUT), lambda i: (0, 0)),
            pl.BlockSpec((D, OUT), lambda i: (0, 0)),
            pl.BlockSpec((1, OUT), lambda i: (0, 0)),
        ],
        out_specs=pl.BlockSpec((ROWB, OUT), lambda i: (i, 0)),
        out_shape=jax.ShapeDtypeStruct((N, OUT), jnp.float32),
    )(h, p0, p1, w1t, w2t, b2d)


def kernel(h, edge_index, edge_w, W, b):
    src = edge_index[0].astype(jnp.int32).reshape(NW, EPW)
    dst = edge_index[1].astype(jnp.int32).reshape(NW, NCHUNK, 1, CHUNK)
    w = edge_w.reshape(NW, NCHUNK, 1, CHUNK).astype(jnp.float32)
    partials = _sc_aggregate(h, src, dst, w)[:, :N, :]
    w1t = W[:, :D].T
    w2t = W[:, D:].T
    return _linear(h, partials[0], partials[1], w1t, w2t, b.reshape(1, OUT))


# speedup vs baseline: 10.7327x; 1.0312x over previous
import jax
import jax.numpy as jnp
from jax import lax
from jax.experimental import pallas as pl
from jax.experimental.pallas import tpu as pltpu
from jax.experimental.pallas import tpu_sc as plsc

N = 10000
E = 320000
D = 128
OUT = 128

NC = 2
NS = 16
NW = NC * NS

NBUF = 3
CHUNK = 80
EPW = E // NW
NCHUNK = EPW // CHUNK
NPAD = 10240
ROWS_PT = NPAD // NS


def _scale_rows(buf, wc):
    def scale_group(g, _):
        w16 = wc[pl.ds(g * 16, 16)]
        for l in range(16):
            wv = lax.gather(
                w16, jnp.full((16, 1), l, jnp.int32),
                lax.GatherDimensionNumbers(offset_dims=(),
                                           collapsed_slice_dims=(0,),
                                           start_index_map=(0,)),
                (1,), mode=lax.GatherScatterMode.PROMISE_IN_BOUNDS)
            r = g * 16 + l
            for u in range(D // 16):
                buf[r, pl.ds(u * 16, 16)] = buf[r, pl.ds(u * 16, 16)] * wv
        return 0
    lax.fori_loop(0, CHUNK // 16, scale_group, 0)


def _sc_agg_body(h_hbm, src_hbm, dst_hbm, w_hbm, out_hbm,
                 src_v, dst_c, wc_v, buf_v, agg_sh,
                 sems, wsems, dsems, ssems, zsem):
    cid = lax.axis_index("c")
    sid = lax.axis_index("s")
    wid = cid * NS + sid

    def start_fetch(j, b):
        pltpu.async_copy(w_hbm.at[wid, j], wc_v.at[b], wsems.at[b])
        pltpu.async_copy(dst_hbm.at[wid, j], dst_c.at[b], dsems.at[b])
        pltpu.async_copy(h_hbm.at[src_v.at[pl.ds(j * CHUNK, CHUNK)]],
                         buf_v.at[b], sems.at[b])

    def wait_fetch(j, b):
        pltpu.make_async_copy(w_hbm.at[wid, j], wc_v.at[b], wsems.at[b]).wait()
        pltpu.make_async_copy(dst_hbm.at[wid, j], dst_c.at[b],
                              dsems.at[b]).wait()
        pltpu.make_async_copy(h_hbm.at[src_v.at[pl.ds(j * CHUNK, CHUNK)]],
                              buf_v.at[b], sems.at[b]).wait()

    def start_scatter(b):
        pltpu.async_copy(buf_v.at[b], agg_sh.at[dst_c.at[b, 0]], ssems.at[b],
                         add=True)

    def wait_scatter(b):
        pltpu.make_async_copy(buf_v.at[b], agg_sh.at[dst_c.at[b, 0]],
                              ssems.at[b]).wait()

    pltpu.sync_copy(src_hbm.at[wid], src_v)
    start_fetch(0, 0)
    start_fetch(1, 1)

    def zero_row(r, _):
        z = jnp.zeros((16,), jnp.float32)
        for u in range(D // 16):
            buf_v[2, r, pl.ds(u * 16, 16)] = z
        return 0
    lax.fori_loop(0, CHUNK, zero_row, 0)
    for t in range(ROWS_PT // CHUNK):
        pltpu.async_copy(buf_v.at[2],
                         agg_sh.at[pl.ds(sid * ROWS_PT + t * CHUNK, CHUNK)],
                         zsem)
    for t in range(ROWS_PT // CHUNK):
        pltpu.make_async_copy(buf_v.at[2],
                              agg_sh.at[pl.ds(sid * ROWS_PT + t * CHUNK,
                                              CHUNK)],
                              zsem).wait()
    plsc.subcore_barrier()

    def step(j, b, fetch_ahead, drain_prev):
        wait_fetch(j, b)
        _scale_rows(buf_v.at[b], wc_v.at[b, 0])
        if drain_prev:
            wait_scatter((b + 2) % NBUF)
        if fetch_ahead:
            start_fetch(j + 2, (b + 2) % NBUF)
        start_scatter(b)

    def do_triple(jt, _):
        for b in range(NBUF):
            j = jt * NBUF + b
            wait_fetch(j, b)
            _scale_rows(buf_v.at[b], wc_v.at[b, 0])

            @pl.when(j >= 1)
            def _():
                wait_scatter((b + 2) % NBUF)
            start_fetch(j + 2, (b + 2) % NBUF)
            start_scatter(b)
        return 0
    lax.fori_loop(0, (NCHUNK - 2) // NBUF, do_triple, 0)
    step(NCHUNK - 2, 0, False, True)
    step(NCHUNK - 1, 1, False, True)
    wait_scatter(1)

    plsc.subcore_barrier()
    pltpu.sync_copy(agg_sh.at[pl.ds(sid * ROWS_PT, ROWS_PT)],
                    out_hbm.at[cid, pl.ds(sid * ROWS_PT, ROWS_PT)])


@jax.jit
def _sc_aggregate(h, src, dst, w):
    mesh = plsc.VectorSubcoreMesh(core_axis_name="c", subcore_axis_name="s")
    return pl.kernel(
        _sc_agg_body,
        out_type=jax.ShapeDtypeStruct((NC, NPAD, D), jnp.float32),
        mesh=mesh,
        scratch_types=[
            pltpu.VMEM((EPW,), jnp.int32),
            pltpu.VMEM((NBUF, 1, CHUNK), jnp.int32),
            pltpu.VMEM((NBUF, 1, CHUNK), jnp.float32),
            pltpu.VMEM((NBUF, CHUNK, D), jnp.float32),
            pltpu.VMEM_SHARED((NPAD, D), jnp.float32),
            pltpu.SemaphoreType.DMA((NBUF,)),
            pltpu.SemaphoreType.DMA((NBUF,)),
            pltpu.SemaphoreType.DMA((NBUF,)),
            pltpu.SemaphoreType.DMA((NBUF,)),
            pltpu.SemaphoreType.DMA,
        ],
    )(h, src, dst, w)


ROWB = 400


def _linear_body(h_b, p_b, w1t_b, w2t_b, b_b, out_b):
    agg = p_b[0] + p_b[1]
    acc = jnp.dot(h_b[...], w1t_b[...], preferred_element_type=jnp.float32)
    acc += jnp.dot(agg, w2t_b[...], preferred_element_type=jnp.float32)
    out_b[...] = acc + b_b[...]


@jax.jit
def _linear(h, partials, w1t, w2t, b2d):
    grid = (N // ROWB,)
    return pl.pallas_call(
        _linear_body,
        grid=grid,
        in_specs=[
            pl.BlockSpec((ROWB, D), lambda i: (i, 0)),
            pl.BlockSpec((NC, ROWB, D), lambda i: (0, i, 0)),
            pl.BlockSpec((D, OUT), lambda i: (0, 0)),
            pl.BlockSpec((D, OUT), lambda i: (0, 0)),
            pl.BlockSpec((1, OUT), lambda i: (0, 0)),
        ],
        out_specs=pl.BlockSpec((ROWB, OUT), lambda i: (i, 0)),
        out_shape=jax.ShapeDtypeStruct((N, OUT), jnp.float32),
    )(h, partials, w1t, w2t, b2d)


def kernel(h, edge_index, edge_w, W, b):
    src = edge_index[0].astype(jnp.int32).reshape(NW, EPW)
    dst = edge_index[1].astype(jnp.int32).reshape(NW, NCHUNK, 1, CHUNK)
    w = edge_w.reshape(NW, NCHUNK, 1, CHUNK).astype(jnp.float32)
    partials = _sc_aggregate(h, src, dst, w)
    w1t = W[:, :D].T
    w2t = W[:, D:].T
    return _linear(h, partials, w1t, w2t, b.reshape(1, OUT))
